# baseline (speedup 1.0000x reference)
import jax
import jax.numpy as jnp
from jax import lax
from jax.experimental import pallas as pl
from jax.experimental.pallas import tpu as pltpu
from jax.experimental.pallas import tpu_sc as plsc

N = 2048
C = 128
CK = 64
K = 16
S = 32
NG = 8
NCOMBO = 4
RQ = 128
RSB = 2048
QB = RSB // S
GW = 128


def _prep_body(fq_ref, pq_ref, fr_ref, pr_ref, wq_ref, wr_ref, wp_ref,
               bq_ref, br_ref, q_ref, r_ref):
    fq = fq_ref[0]
    fr = fr_ref[0]
    pq = pq_ref[0]
    pr = pr_ref[0]
    posq = (pq[:, 0:1] * wp_ref[0:1, :] + pq[:, 1:2] * wp_ref[1:2, :]
            + pq[:, 2:3] * wp_ref[2:3, :])
    posr = (pr[:, 0:1] * wp_ref[0:1, :] + pr[:, 1:2] * wp_ref[1:2, :]
            + pr[:, 2:3] * wp_ref[2:3, :])
    q_ref[0] = jnp.dot(fq, wq_ref[...], preferred_element_type=jnp.float32) \
        + posq + bq_ref[...]
    r_ref[0] = jnp.dot(fr, wr_ref[...], preferred_element_type=jnp.float32) \
        - posr + br_ref[...]


def _prep(Fq, Pq, Fr, Pr, WqT, WrT, WpT, bq, br):
    RB = 512
    nc = Fq.shape[0]
    grid = (nc, N // RB)
    return pl.pallas_call(
        _prep_body,
        grid=grid,
        compiler_params=pltpu.CompilerParams(
            dimension_semantics=("parallel", "parallel")),
        in_specs=[
            pl.BlockSpec((1, RB, C), lambda c, n: (c, n, 0)),
            pl.BlockSpec((1, RB, 3), lambda c, n: (c, n, 0)),
            pl.BlockSpec((1, RB, C), lambda c, n: (c, n, 0)),
            pl.BlockSpec((1, RB, 3), lambda c, n: (c, n, 0)),
            pl.BlockSpec((C, C), lambda c, n: (0, 0)),
            pl.BlockSpec((C, C), lambda c, n: (0, 0)),
            pl.BlockSpec((3, C), lambda c, n: (0, 0)),
            pl.BlockSpec((1, C), lambda c, n: (0, 0)),
            pl.BlockSpec((1, C), lambda c, n: (0, 0)),
        ],
        out_specs=[
            pl.BlockSpec((1, RB, C), lambda c, n: (c, n, 0)),
            pl.BlockSpec((1, RB, C), lambda c, n: (c, n, 0)),
        ],
        out_shape=[
            jax.ShapeDtypeStruct((nc, N, C), jnp.float32),
            jax.ShapeDtypeStruct((nc, N, C), jnp.float32),
        ],
    )(Fq, Pq, Fr, Pr, WqT, WrT, WpT, bq, br)


NCH = 128
CW = N // NCH
RND = 5


def _extract_min16(d, acc, lane_off, lane32):
    iota_j = lax.broadcasted_iota(jnp.int32, d.shape, 1)
    for k in range(K):
        idx = jnp.argmin(d, axis=1).astype(jnp.int32)[:, None]
        acc = jnp.where(lane32 == (k + lane_off), idx, acc)
        d = jnp.where(iota_j == idx, jnp.float32(jnp.inf), d)
    return acc


def _fold_min(x):
    w = x.shape[1]
    while w > NCH:
        w //= 2
        x = jnp.minimum(x[:, :w], x[:, w:])
    return x


def _extract16_fast(d, acc, lane_off, lane32):
    iota_j = lax.broadcasted_iota(jnp.int32, (RQ, N), 1)
    vals_l, idxs_l = [], []
    for _ in range(RND):
        cm = _fold_min(d)
        cmt = jnp.tile(cm, (1, CW))
        cand = jnp.where(d == cmt, iota_j, jnp.int32(N))
        am = _fold_min(cand)
        amt = jnp.tile(am, (1, CW))
        d = jnp.where(cand == amt, jnp.float32(jnp.inf), d)
        vals_l.append(cm)
        idxs_l.append(am)
    vals = jnp.concatenate(vals_l, axis=1)
    idxs = jnp.concatenate(idxs_l, axis=1)
    for k in range(K):
        m = jnp.min(vals, axis=1, keepdims=True)
        c2 = jnp.where(vals == m, idxs, jnp.int32(N))
        jmin = jnp.min(c2, axis=1, keepdims=True)
        acc = jnp.where(lane32 == (k + lane_off), jmin, acc)
        vals = jnp.where((vals == m) & (idxs == jmin),
                         jnp.float32(jnp.inf), vals)
    exhausted = jnp.any(jnp.isinf(vals[:, (RND - 1) * NCH:]))
    return acc, exhausted


def _topk(Aq, Bc, Xq, Xc, base):
    nc = Aq.shape[0]

    def body(aq_ref, bc_ref, xq_ref, xc_ref, out_ref):
        combo = pl.program_id(0)
        xq = xq_ref[0]
        dsq = jnp.zeros((RQ, N), jnp.float32)
        for cdim in range(3):
            diff = xq[:, cdim:cdim + 1] - xc_ref[0, cdim:cdim + 1, :]
            dsq = dsq + diff * diff
        dpoint = jnp.sqrt(dsq)
        a = aq_ref[0]
        an = a / (jnp.sqrt(jnp.sum(a * a, axis=1, keepdims=True)) + 1e-8)
        b = bc_ref[0]
        bn = b / (jnp.sqrt(jnp.sum(b * b, axis=1, keepdims=True)) + 1e-8)
        sim = lax.dot_general(an, bn, (((1,), (1,)), ((), ())),
                              preferred_element_type=jnp.float32)
        dfeat = 1.0 - sim
        lane32 = lax.broadcasted_iota(jnp.int32, (RQ, S), 1)
        acc = jnp.zeros((RQ, S), jnp.int32)
        acc, bad_p = _extract16_fast(dpoint, acc, 0, lane32)
        acc, bad_f = _extract16_fast(dfeat, acc, K, lane32)
        out_ref[0] = acc + (combo + base) * N

        @pl.when(bad_p | bad_f)
        def _():
            acc2 = jnp.zeros((RQ, S), jnp.int32)
            acc2 = _extract_min16(dpoint, acc2, 0, lane32)
            acc2 = _extract_min16(dfeat, acc2, K, lane32)
            out_ref[0] = acc2 + (combo + base) * N

    grid = (nc, N // RQ)
    return pl.pallas_call(
        body,
        grid=grid,
        compiler_params=pltpu.CompilerParams(
            dimension_semantics=("parallel", "parallel")),
        in_specs=[
            pl.BlockSpec((1, RQ, CK), lambda c, n: (c, n, 0)),
            pl.BlockSpec((1, N, CK), lambda c, n: (c, 0, 0)),
            pl.BlockSpec((1, RQ, 3), lambda c, n: (c, n, 0)),
            pl.BlockSpec((1, 3, N), lambda c, n: (c, 0, 0)),
        ],
        out_specs=pl.BlockSpec((1, RQ, S), lambda c, n: (c, n, 0)),
        out_shape=jax.ShapeDtypeStruct((nc, N, S), jnp.int32),
    )(Aq, Bc, Xq, Xc)


def _sc_gather(qflat, idxflat):
    num_idx = idxflat.shape[0]
    idx2 = idxflat.reshape(1, num_idx)
    mesh = plsc.VectorSubcoreMesh(core_axis_name="c", subcore_axis_name="s")

    @pl.kernel(out_type=jax.ShapeDtypeStruct((num_idx, C), jnp.float32),
               mesh=mesh)
    def k(x_hbm, i_hbm, o_hbm):
        def body(i_vmem, o_vmem):
            pltpu.sync_copy(x_hbm.at[i_vmem.at[0]], o_vmem)

        pltpu.emit_pipeline(
            body,
            grid=(num_idx // GW,),
            in_specs=[pl.BlockSpec((1, GW), index_map=lambda i: (0, i))],
            out_specs=[pl.BlockSpec((GW, C), index_map=lambda i: (i, 0))],
            core_axis_name=("c", "s"),
            dimension_semantics=(pltpu.PARALLEL,),
        )(i_hbm, o_hbm)

    return k(qflat, idx2)


def _x0_block(g_ref, r_ref):
    g = g_ref[0]
    r = r_ref[0]
    rrep = jnp.broadcast_to(r[:, None, :], (QB, S, C)).reshape(RSB, C)
    return g + rrep


def _accum_stats(x, sum_ref, sumsq_ref, nblk_id):
    ps = jnp.sum(x, axis=0, keepdims=True)
    psq = jnp.sum(x * x, axis=0, keepdims=True)

    @pl.when(nblk_id == 0)
    def _():
        sum_ref[0] = jnp.zeros((1, C), jnp.float32)
        sumsq_ref[0] = jnp.zeros((1, C), jnp.float32)

    sum_ref[0] += ps
    sumsq_ref[0] += psq


def _stats0_body(g_ref, r_ref, sum_ref, sumsq_ref):
    x0 = _x0_block(g_ref, r_ref)
    _accum_stats(x0, sum_ref, sumsq_ref, pl.program_id(1))


def _stats0(Gact, Rt):
    nc = Gact.shape[0]
    grid = (nc, (N * S) // RSB)
    return pl.pallas_call(
        _stats0_body,
        grid=grid,
        compiler_params=pltpu.CompilerParams(
            dimension_semantics=("parallel", "arbitrary")),
        in_specs=[
            pl.BlockSpec((1, RSB, C), lambda c, n: (c, n, 0)),
            pl.BlockSpec((1, QB, C), lambda c, n: (c, n, 0)),
        ],
        out_specs=[
            pl.BlockSpec((1, 1, C), lambda c, n: (c, 0, 0)),
            pl.BlockSpec((1, 1, C), lambda c, n: (c, 0, 0)),
        ],
        out_shape=[
            jax.ShapeDtypeStruct((nc, 1, C), jnp.float32),
            jax.ShapeDtypeStruct((nc, 1, C), jnp.float32),
        ],
    )(Gact, Rt)


def _lrelu(x):
    return jnp.where(x >= 0, x, 0.1 * x)


def _gn_coeffs_inkernel(sum_ref, sumsq_ref, ind_ref, indT_ref, gb_ref):
    M = jnp.float32(N * S * (C // NG))
    s = sum_ref[0]
    sq = sumsq_ref[0]
    gs = jnp.dot(s, ind_ref[...], preferred_element_type=jnp.float32)
    gsq = jnp.dot(sq, ind_ref[...], preferred_element_type=jnp.float32)
    mean_c = jnp.dot(gs, indT_ref[...],
                     preferred_element_type=jnp.float32) / M
    msq_c = jnp.dot(gsq, indT_ref[...],
                    preferred_element_type=jnp.float32) / M
    var_c = msq_c - mean_c * mean_c
    inv_c = jax.lax.rsqrt(var_c + 1e-5)
    scale = inv_c * gb_ref[0:1, :]
    shift = gb_ref[1:2, :] - mean_c * scale
    return scale, shift


def _mlp0_body(g_ref, r_ref, s0_ref, sq0_ref, ind_ref, indT_ref, gb_ref,
               w_ref, b_ref, x1_ref, sum_ref, sumsq_ref):
    sc, sh = _gn_coeffs_inkernel(s0_ref, sq0_ref, ind_ref, indT_ref, gb_ref)
    x0 = _x0_block(g_ref, r_ref)
    y0 = _lrelu(x0 * sc + sh)
    x1 = jnp.dot(y0, w_ref[...], preferred_element_type=jnp.float32) \
        + b_ref[...]
    x1_ref[0] = x1
    _accum_stats(x1, sum_ref, sumsq_ref, pl.program_id(1))


def _mlp0(Gact, Rt, sum0, sumsq0, Ind, IndT, gb0, W1T, b1):
    nc = Gact.shape[0]
    grid = (nc, (N * S) // RSB)
    return pl.pallas_call(
        _mlp0_body,
        grid=grid,
        compiler_params=pltpu.CompilerParams(
            dimension_semantics=("parallel", "arbitrary")),
        in_specs=[
            pl.BlockSpec((1, RSB, C), lambda c, n: (c, n, 0)),
            pl.BlockSpec((1, QB, C), lambda c, n: (c, n, 0)),
            pl.BlockSpec((1, 1, C), lambda c, n: (c, 0, 0)),
            pl.BlockSpec((1, 1, C), lambda c, n: (c, 0, 0)),
            pl.BlockSpec((C, NG), lambda c, n: (0, 0)),
            pl.BlockSpec((NG, C), lambda c, n: (0, 0)),
            pl.BlockSpec((2, C), lambda c, n: (0, 0)),
            pl.BlockSpec((C, C), lambda c, n: (0, 0)),
            pl.BlockSpec((1, C), lambda c, n: (0, 0)),
        ],
        out_specs=[
            pl.BlockSpec((1, RSB, C), lambda c, n: (c, n, 0)),
            pl.BlockSpec((1, 1, C), lambda c, n: (c, 0, 0)),
            pl.BlockSpec((1, 1, C), lambda c, n: (c, 0, 0)),
        ],
        out_shape=[
            jax.ShapeDtypeStruct((nc, N * S, C), jnp.float32),
            jax.ShapeDtypeStruct((nc, 1, C), jnp.float32),
            jax.ShapeDtypeStruct((nc, 1, C), jnp.float32),
        ],
    )(Gact, Rt, sum0, sumsq0, Ind, IndT, gb0, W1T, b1)


def _mlp1_body(x_ref, s1_ref, sq1_ref, ind_ref, indT_ref, gb_ref,
               w_ref, b_ref, x2_ref, sum_ref, sumsq_ref):
    sc, sh = _gn_coeffs_inkernel(s1_ref, sq1_ref, ind_ref, indT_ref, gb_ref)
    y = _lrelu(x_ref[0] * sc + sh)
    x2 = jnp.dot(y, w_ref[...], preferred_element_type=jnp.float32) \
        + b_ref[...]
    x2_ref[0] = x2
    _accum_stats(x2, sum_ref, sumsq_ref, pl.program_id(1))


def _mlp1(X1, sum1, sumsq1, Ind, IndT, gb1, W2T, b2):
    nc = X1.shape[0]
    grid = (nc, (N * S) // RSB)
    return pl.pallas_call(
        _mlp1_body,
        grid=grid,
        compiler_params=pltpu.CompilerParams(
            dimension_semantics=("parallel", "arbitrary")),
        in_specs=[
            pl.BlockSpec((1, RSB, C), lambda c, n: (c, n, 0)),
            pl.BlockSpec((1, 1, C), lambda c, n: (c, 0, 0)),
            pl.BlockSpec((1, 1, C), lambda c, n: (c, 0, 0)),
            pl.BlockSpec((C, NG), lambda c, n: (0, 0)),
            pl.BlockSpec((NG, C), lambda c, n: (0, 0)),
            pl.BlockSpec((2, C), lambda c, n: (0, 0)),
            pl.BlockSpec((C, C), lambda c, n: (0, 0)),
            pl.BlockSpec((1, C), lambda c, n: (0, 0)),
        ],
        out_specs=[
            pl.BlockSpec((1, RSB, C), lambda c, n: (c, n, 0)),
            pl.BlockSpec((1, 1, C), lambda c, n: (c, 0, 0)),
            pl.BlockSpec((1, 1, C), lambda c, n: (c, 0, 0)),
        ],
        out_shape=[
            jax.ShapeDtypeStruct((nc, N * S, C), jnp.float32),
            jax.ShapeDtypeStruct((nc, 1, C), jnp.float32),
            jax.ShapeDtypeStruct((nc, 1, C), jnp.float32),
        ],
    )(X1, sum1, sumsq1, Ind, IndT, gb1, W2T, b2)


def _final_body(x_ref, s2_ref, sq2_ref, ind_ref, indT_ref, gb_ref, out_ref):
    sc, sh = _gn_coeffs_inkernel(s2_ref, sq2_ref, ind_ref, indT_ref, gb_ref)
    y = _lrelu(x_ref[0] * sc + sh)
    out_ref[0] = jnp.max(y.reshape(QB, S, C), axis=1)


def _final(X2, sum2, sumsq2, Ind, IndT, gb2):
    nc = X2.shape[0]
    grid = (nc, (N * S) // RSB)
    return pl.pallas_call(
        _final_body,
        grid=grid,
        compiler_params=pltpu.CompilerParams(
            dimension_semantics=("parallel", "parallel")),
        in_specs=[
            pl.BlockSpec((1, RSB, C), lambda c, n: (c, n, 0)),
            pl.BlockSpec((1, 1, C), lambda c, n: (c, 0, 0)),
            pl.BlockSpec((1, 1, C), lambda c, n: (c, 0, 0)),
            pl.BlockSpec((C, NG), lambda c, n: (0, 0)),
            pl.BlockSpec((NG, C), lambda c, n: (0, 0)),
            pl.BlockSpec((2, C), lambda c, n: (0, 0)),
        ],
        out_specs=pl.BlockSpec((1, QB, C), lambda c, n: (c, n, 0)),
        out_shape=jax.ShapeDtypeStruct((nc, N, C), jnp.float32),
    )(X2, sum2, sumsq2, Ind, IndT, gb2)


def kernel(pc1, pc2, feat1, feat2, knn1, knn2, W_t11, b_t11, W_t22, b_t22,
           W_pos, b_pos, gn0_g, gn0_b, W_m1, b_m1, gn1_g, gn1_b,
           W_m2, b_m2, gn2_g, gn2_b):
    f32 = jnp.float32
    f1T = jnp.transpose(feat1, (0, 2, 1))
    f2T = jnp.transpose(feat2, (0, 2, 1))
    p1T = jnp.transpose(pc1, (0, 2, 1))
    p2T = jnp.transpose(pc2, (0, 2, 1))
    k1T = jnp.transpose(knn1, (0, 2, 1))
    k2T = jnp.transpose(knn2, (0, 2, 1))

    Fq = jnp.concatenate([f2T, f1T], axis=0)
    Pq = jnp.concatenate([p2T, p1T], axis=0)
    Fr = jnp.concatenate([f1T, f2T], axis=0)
    Pr = jnp.concatenate([p1T, p2T], axis=0)
    Aq = jnp.concatenate([k1T, k2T], axis=0)
    Bc = jnp.concatenate([k2T, k1T], axis=0)
    Xq = Pr
    Xc = jnp.concatenate([pc2, pc1], axis=0)

    Q, Rt = _prep(Fq, Pq, Fr, Pr,
                  W_t22.T.astype(f32), W_t11.T.astype(f32),
                  W_pos.T.astype(f32),
                  b_t22.reshape(1, C), b_t11.reshape(1, C) + b_pos.reshape(1, C))
    Qflat = Q.reshape(NCOMBO * N, C)

    W1T = W_m1.T.astype(f32)
    W2T = W_m2.T.astype(f32)
    b1 = b_m1.reshape(1, C)
    b2 = b_m2.reshape(1, C)
    ch = jnp.arange(C) // (C // NG)
    Ind = (ch[:, None] == jnp.arange(NG)[None, :]).astype(f32)
    IndT = Ind.T
    gb0 = jnp.stack([gn0_g, gn0_b]).astype(f32)
    gb1 = jnp.stack([gn1_g, gn1_b]).astype(f32)
    gb2 = jnp.stack([gn2_g, gn2_b]).astype(f32)

    outs = []
    gacts = []
    for p in (0, 1):
        sl = slice(2 * p, 2 * p + 2)
        knn_idx = _topk(Aq[sl], Bc[sl], Xq[sl], Xc[sl], base=2 * p)
        gacts.append(_sc_gather(Qflat, knn_idx.reshape(2 * N * S))
                     .reshape(2, N * S, C))
    for p in (0, 1):
        sl = slice(2 * p, 2 * p + 2)
        Gact = gacts[p]
        Rp = Rt[sl]
        sum0, sumsq0 = _stats0(Gact, Rp)
        X1, sum1, sumsq1 = _mlp0(Gact, Rp, sum0, sumsq0, Ind, IndT, gb0,
                                 W1T, b1)
        X2, sum2, sumsq2 = _mlp1(X1, sum1, sumsq1, Ind, IndT, gb1, W2T, b2)
        outs.append(_final(X2, sum2, sumsq2, Ind, IndT, gb2))

    feat1_new = jnp.transpose(outs[0], (0, 2, 1))
    feat2_new = jnp.transpose(outs[1], (0, 2, 1))
    return (feat1_new, feat2_new)


# speedup vs baseline: 1.0154x; 1.0154x over previous
import jax
import jax.numpy as jnp
from jax import lax
from jax.experimental import pallas as pl
from jax.experimental.pallas import tpu as pltpu
from jax.experimental.pallas import tpu_sc as plsc

N = 2048
C = 128
CK = 64
K = 16
S = 32
NG = 8
NCOMBO = 4
RQ = 128
RSB = 2048
QB = RSB // S
GW = 128


def _prep_body(fq_ref, pq_ref, fr_ref, pr_ref, wq_ref, wr_ref, wp_ref,
               bq_ref, br_ref, q_ref, r_ref):
    fq = fq_ref[0]
    fr = fr_ref[0]
    pq = pq_ref[0]
    pr = pr_ref[0]
    posq = (pq[:, 0:1] * wp_ref[0:1, :] + pq[:, 1:2] * wp_ref[1:2, :]
            + pq[:, 2:3] * wp_ref[2:3, :])
    posr = (pr[:, 0:1] * wp_ref[0:1, :] + pr[:, 1:2] * wp_ref[1:2, :]
            + pr[:, 2:3] * wp_ref[2:3, :])
    q_ref[0] = jnp.dot(fq, wq_ref[...], preferred_element_type=jnp.float32) \
        + posq + bq_ref[...]
    r_ref[0] = jnp.dot(fr, wr_ref[...], preferred_element_type=jnp.float32) \
        - posr + br_ref[...]


def _prep(Fq, Pq, Fr, Pr, WqT, WrT, WpT, bq, br):
    RB = 512
    nc = Fq.shape[0]
    grid = (nc, N // RB)
    return pl.pallas_call(
        _prep_body,
        grid=grid,
        compiler_params=pltpu.CompilerParams(
            dimension_semantics=("parallel", "parallel")),
        in_specs=[
            pl.BlockSpec((1, RB, C), lambda c, n: (c, n, 0)),
            pl.BlockSpec((1, RB, 3), lambda c, n: (c, n, 0)),
            pl.BlockSpec((1, RB, C), lambda c, n: (c, n, 0)),
            pl.BlockSpec((1, RB, 3), lambda c, n: (c, n, 0)),
            pl.BlockSpec((C, C), lambda c, n: (0, 0)),
            pl.BlockSpec((C, C), lambda c, n: (0, 0)),
            pl.BlockSpec((3, C), lambda c, n: (0, 0)),
            pl.BlockSpec((1, C), lambda c, n: (0, 0)),
            pl.BlockSpec((1, C), lambda c, n: (0, 0)),
        ],
        out_specs=[
            pl.BlockSpec((1, RB, C), lambda c, n: (c, n, 0)),
            pl.BlockSpec((1, RB, C), lambda c, n: (c, n, 0)),
        ],
        out_shape=[
            jax.ShapeDtypeStruct((nc, N, C), jnp.float32),
            jax.ShapeDtypeStruct((nc, N, C), jnp.float32),
        ],
    )(Fq, Pq, Fr, Pr, WqT, WrT, WpT, bq, br)


NCH = 128
CW = N // NCH
RND = 4


def _extract_min16(d, acc, lane_off, lane32):
    iota_j = lax.broadcasted_iota(jnp.int32, d.shape, 1)
    for k in range(K):
        idx = jnp.argmin(d, axis=1).astype(jnp.int32)[:, None]
        acc = jnp.where(lane32 == (k + lane_off), idx, acc)
        d = jnp.where(iota_j == idx, jnp.float32(jnp.inf), d)
    return acc


def _fold_min(x):
    w = x.shape[1]
    while w > NCH:
        w //= 2
        x = jnp.minimum(x[:, :w], x[:, w:])
    return x


def _extract16_fast(d, acc, lane_off, lane32):
    iota_j = lax.broadcasted_iota(jnp.int32, (RQ, N), 1)
    vals_l, idxs_l = [], []
    for _ in range(RND):
        cm = _fold_min(d)
        cmt = jnp.tile(cm, (1, CW))
        cand = jnp.where(d == cmt, iota_j, jnp.int32(N))
        am = _fold_min(cand)
        amt = jnp.tile(am, (1, CW))
        d = jnp.where(cand == amt, jnp.float32(jnp.inf), d)
        vals_l.append(cm)
        idxs_l.append(am)
    vals = jnp.concatenate(vals_l, axis=1)
    idxs = jnp.concatenate(idxs_l, axis=1)
    for k in range(K):
        m = jnp.min(vals, axis=1, keepdims=True)
        c2 = jnp.where(vals == m, idxs, jnp.int32(N))
        jmin = jnp.min(c2, axis=1, keepdims=True)
        acc = jnp.where(lane32 == (k + lane_off), jmin, acc)
        vals = jnp.where((vals == m) & (idxs == jmin),
                         jnp.float32(jnp.inf), vals)
    exhausted = jnp.any(jnp.isinf(vals[:, (RND - 1) * NCH:]))
    return acc, exhausted


def _topk(Aq, Bc, Xq, Xc, base):
    nc = Aq.shape[0]

    def body(aq_ref, bc_ref, xq_ref, xc_ref, out_ref):
        combo = pl.program_id(0)
        xq = xq_ref[0]
        dsq = jnp.zeros((RQ, N), jnp.float32)
        for cdim in range(3):
            diff = xq[:, cdim:cdim + 1] - xc_ref[0, cdim:cdim + 1, :]
            dsq = dsq + diff * diff
        dpoint = jnp.sqrt(dsq)
        a = aq_ref[0]
        an = a / (jnp.sqrt(jnp.sum(a * a, axis=1, keepdims=True)) + 1e-8)
        b = bc_ref[0]
        bn = b / (jnp.sqrt(jnp.sum(b * b, axis=1, keepdims=True)) + 1e-8)
        sim = lax.dot_general(an, bn, (((1,), (1,)), ((), ())),
                              preferred_element_type=jnp.float32)
        dfeat = 1.0 - sim
        lane32 = lax.broadcasted_iota(jnp.int32, (RQ, S), 1)
        acc = jnp.zeros((RQ, S), jnp.int32)
        acc, bad_p = _extract16_fast(dpoint, acc, 0, lane32)
        acc, bad_f = _extract16_fast(dfeat, acc, K, lane32)
        out_ref[0] = acc + (combo + base) * N

        @pl.when(bad_p | bad_f)
        def _():
            acc2 = jnp.zeros((RQ, S), jnp.int32)
            acc2 = _extract_min16(dpoint, acc2, 0, lane32)
            acc2 = _extract_min16(dfeat, acc2, K, lane32)
            out_ref[0] = acc2 + (combo + base) * N

    grid = (nc, N // RQ)
    return pl.pallas_call(
        body,
        grid=grid,
        compiler_params=pltpu.CompilerParams(
            dimension_semantics=("parallel", "parallel")),
        in_specs=[
            pl.BlockSpec((1, RQ, CK), lambda c, n: (c, n, 0)),
            pl.BlockSpec((1, N, CK), lambda c, n: (c, 0, 0)),
            pl.BlockSpec((1, RQ, 3), lambda c, n: (c, n, 0)),
            pl.BlockSpec((1, 3, N), lambda c, n: (c, 0, 0)),
        ],
        out_specs=pl.BlockSpec((1, RQ, S), lambda c, n: (c, n, 0)),
        out_shape=jax.ShapeDtypeStruct((nc, N, S), jnp.int32),
    )(Aq, Bc, Xq, Xc)


def _sc_gather(qflat, idxflat):
    num_idx = idxflat.shape[0]
    idx2 = idxflat.reshape(1, num_idx)
    mesh = plsc.VectorSubcoreMesh(core_axis_name="c", subcore_axis_name="s")

    @pl.kernel(out_type=jax.ShapeDtypeStruct((num_idx, C), jnp.float32),
               mesh=mesh)
    def k(x_hbm, i_hbm, o_hbm):
        def body(i_vmem, o_vmem):
            pltpu.sync_copy(x_hbm.at[i_vmem.at[0]], o_vmem)

        pltpu.emit_pipeline(
            body,
            grid=(num_idx // GW,),
            in_specs=[pl.BlockSpec((1, GW), index_map=lambda i: (0, i))],
            out_specs=[pl.BlockSpec((GW, C), index_map=lambda i: (i, 0))],
            core_axis_name=("c", "s"),
            dimension_semantics=(pltpu.PARALLEL,),
        )(i_hbm, o_hbm)

    return k(qflat, idx2)


def _x0_block(g_ref, r_ref):
    g = g_ref[0]
    r = r_ref[0]
    rrep = jnp.broadcast_to(r[:, None, :], (QB, S, C)).reshape(RSB, C)
    return g + rrep


def _accum_stats(x, sum_ref, sumsq_ref, nblk_id):
    ps = jnp.sum(x, axis=0, keepdims=True)
    psq = jnp.sum(x * x, axis=0, keepdims=True)

    @pl.when(nblk_id == 0)
    def _():
        sum_ref[0] = jnp.zeros((1, C), jnp.float32)
        sumsq_ref[0] = jnp.zeros((1, C), jnp.float32)

    sum_ref[0] += ps
    sumsq_ref[0] += psq


def _stats0_body(g_ref, r_ref, sum_ref, sumsq_ref):
    x0 = _x0_block(g_ref, r_ref)
    _accum_stats(x0, sum_ref, sumsq_ref, pl.program_id(1))


def _stats0(Gact, Rt):
    nc = Gact.shape[0]
    grid = (nc, (N * S) // RSB)
    return pl.pallas_call(
        _stats0_body,
        grid=grid,
        compiler_params=pltpu.CompilerParams(
            dimension_semantics=("parallel", "arbitrary")),
        in_specs=[
            pl.BlockSpec((1, RSB, C), lambda c, n: (c, n, 0)),
            pl.BlockSpec((1, QB, C), lambda c, n: (c, n, 0)),
        ],
        out_specs=[
            pl.BlockSpec((1, 1, C), lambda c, n: (c, 0, 0)),
            pl.BlockSpec((1, 1, C), lambda c, n: (c, 0, 0)),
        ],
        out_shape=[
            jax.ShapeDtypeStruct((nc, 1, C), jnp.float32),
            jax.ShapeDtypeStruct((nc, 1, C), jnp.float32),
        ],
    )(Gact, Rt)


def _lrelu(x):
    return jnp.where(x >= 0, x, 0.1 * x)


def _mlp0_body(g_ref, r_ref, sc_ref, sh_ref, w_ref, b_ref,
               x1_ref, sum_ref, sumsq_ref):
    x0 = _x0_block(g_ref, r_ref)
    y0 = _lrelu(x0 * sc_ref[0] + sh_ref[0])
    x1 = jnp.dot(y0, w_ref[...], preferred_element_type=jnp.float32) \
        + b_ref[...]
    x1_ref[0] = x1
    _accum_stats(x1, sum_ref, sumsq_ref, pl.program_id(1))


def _mlp0(Gact, Rt, scale0, shift0, W1T, b1):
    nc = Gact.shape[0]
    grid = (nc, (N * S) // RSB)
    return pl.pallas_call(
        _mlp0_body,
        grid=grid,
        compiler_params=pltpu.CompilerParams(
            dimension_semantics=("parallel", "arbitrary")),
        in_specs=[
            pl.BlockSpec((1, RSB, C), lambda c, n: (c, n, 0)),
            pl.BlockSpec((1, QB, C), lambda c, n: (c, n, 0)),
            pl.BlockSpec((1, 1, C), lambda c, n: (c, 0, 0)),
            pl.BlockSpec((1, 1, C), lambda c, n: (c, 0, 0)),
            pl.BlockSpec((C, C), lambda c, n: (0, 0)),
            pl.BlockSpec((1, C), lambda c, n: (0, 0)),
        ],
        out_specs=[
            pl.BlockSpec((1, RSB, C), lambda c, n: (c, n, 0)),
            pl.BlockSpec((1, 1, C), lambda c, n: (c, 0, 0)),
            pl.BlockSpec((1, 1, C), lambda c, n: (c, 0, 0)),
        ],
        out_shape=[
            jax.ShapeDtypeStruct((nc, N * S, C), jnp.float32),
            jax.ShapeDtypeStruct((nc, 1, C), jnp.float32),
            jax.ShapeDtypeStruct((nc, 1, C), jnp.float32),
        ],
    )(Gact, Rt, scale0, shift0, W1T, b1)


def _mlp1_body(x_ref, sc_ref, sh_ref, w_ref, b_ref,
               x2_ref, sum_ref, sumsq_ref):
    y = _lrelu(x_ref[0] * sc_ref[0] + sh_ref[0])
    x2 = jnp.dot(y, w_ref[...], preferred_element_type=jnp.float32) \
        + b_ref[...]
    x2_ref[0] = x2
    _accum_stats(x2, sum_ref, sumsq_ref, pl.program_id(1))


def _mlp1(X1, scale1, shift1, W2T, b2):
    nc = X1.shape[0]
    grid = (nc, (N * S) // RSB)
    return pl.pallas_call(
        _mlp1_body,
        grid=grid,
        compiler_params=pltpu.CompilerParams(
            dimension_semantics=("parallel", "arbitrary")),
        in_specs=[
            pl.BlockSpec((1, RSB, C), lambda c, n: (c, n, 0)),
            pl.BlockSpec((1, 1, C), lambda c, n: (c, 0, 0)),
            pl.BlockSpec((1, 1, C), lambda c, n: (c, 0, 0)),
            pl.BlockSpec((C, C), lambda c, n: (0, 0)),
            pl.BlockSpec((1, C), lambda c, n: (0, 0)),
        ],
        out_specs=[
            pl.BlockSpec((1, RSB, C), lambda c, n: (c, n, 0)),
            pl.BlockSpec((1, 1, C), lambda c, n: (c, 0, 0)),
            pl.BlockSpec((1, 1, C), lambda c, n: (c, 0, 0)),
        ],
        out_shape=[
            jax.ShapeDtypeStruct((nc, N * S, C), jnp.float32),
            jax.ShapeDtypeStruct((nc, 1, C), jnp.float32),
            jax.ShapeDtypeStruct((nc, 1, C), jnp.float32),
        ],
    )(X1, scale1, shift1, W2T, b2)


def _final_body(x_ref, sc_ref, sh_ref, out_ref):
    y = _lrelu(x_ref[0] * sc_ref[0] + sh_ref[0])
    out_ref[0] = jnp.max(y.reshape(QB, S, C), axis=1)


def _final(X2, scale2, shift2):
    nc = X2.shape[0]
    grid = (nc, (N * S) // RSB)
    return pl.pallas_call(
        _final_body,
        grid=grid,
        compiler_params=pltpu.CompilerParams(
            dimension_semantics=("parallel", "parallel")),
        in_specs=[
            pl.BlockSpec((1, RSB, C), lambda c, n: (c, n, 0)),
            pl.BlockSpec((1, 1, C), lambda c, n: (c, 0, 0)),
            pl.BlockSpec((1, 1, C), lambda c, n: (c, 0, 0)),
        ],
        out_specs=pl.BlockSpec((1, QB, C), lambda c, n: (c, n, 0)),
        out_shape=jax.ShapeDtypeStruct((nc, N, C), jnp.float32),
    )(X2, scale2, shift2)


def _gn_coeffs(sums, sumsqs, gamma, beta):
    nc = sums.shape[0]
    M = float(N * S * (C // NG))
    gsum = sums.reshape(nc, NG, C // NG).sum(axis=2)
    gsumsq = sumsqs.reshape(nc, NG, C // NG).sum(axis=2)
    mean = gsum / M
    var = gsumsq / M - mean * mean
    inv = 1.0 / jnp.sqrt(var + 1e-5)
    mean_c = jnp.repeat(mean, C // NG, axis=1)
    inv_c = jnp.repeat(inv, C // NG, axis=1)
    scale = inv_c * gamma[None, :]
    shift = beta[None, :] - mean_c * scale
    return scale[:, None, :], shift[:, None, :]


def kernel(pc1, pc2, feat1, feat2, knn1, knn2, W_t11, b_t11, W_t22, b_t22,
           W_pos, b_pos, gn0_g, gn0_b, W_m1, b_m1, gn1_g, gn1_b,
           W_m2, b_m2, gn2_g, gn2_b):
    f32 = jnp.float32
    f1T = jnp.transpose(feat1, (0, 2, 1))
    f2T = jnp.transpose(feat2, (0, 2, 1))
    p1T = jnp.transpose(pc1, (0, 2, 1))
    p2T = jnp.transpose(pc2, (0, 2, 1))
    k1T = jnp.transpose(knn1, (0, 2, 1))
    k2T = jnp.transpose(knn2, (0, 2, 1))

    Fq = jnp.concatenate([f2T, f1T], axis=0)
    Pq = jnp.concatenate([p2T, p1T], axis=0)
    Fr = jnp.concatenate([f1T, f2T], axis=0)
    Pr = jnp.concatenate([p1T, p2T], axis=0)
    Aq = jnp.concatenate([k1T, k2T], axis=0)
    Bc = jnp.concatenate([k2T, k1T], axis=0)
    Xq = Pr
    Xc = jnp.concatenate([pc2, pc1], axis=0)

    Q, Rt = _prep(Fq, Pq, Fr, Pr,
                  W_t22.T.astype(f32), W_t11.T.astype(f32),
                  W_pos.T.astype(f32),
                  b_t22.reshape(1, C), b_t11.reshape(1, C) + b_pos.reshape(1, C))
    Qflat = Q.reshape(NCOMBO * N, C)

    W1T = W_m1.T.astype(f32)
    W2T = W_m2.T.astype(f32)
    b1 = b_m1.reshape(1, C)
    b2 = b_m2.reshape(1, C)

    outs = []
    gacts = []
    for p in (0, 1):
        sl = slice(2 * p, 2 * p + 2)
        knn_idx = _topk(Aq[sl], Bc[sl], Xq[sl], Xc[sl], base=2 * p)
        gacts.append(_sc_gather(Qflat, knn_idx.reshape(2 * N * S))
                     .reshape(2, N * S, C))
    for p in (0, 1):
        sl = slice(2 * p, 2 * p + 2)
        Gact = gacts[p]
        Rp = Rt[sl]
        sum0, sumsq0 = _stats0(Gact, Rp)
        sc0, sh0 = _gn_coeffs(sum0[:, 0, :], sumsq0[:, 0, :], gn0_g, gn0_b)
        X1, sum1, sumsq1 = _mlp0(Gact, Rp, sc0, sh0, W1T, b1)
        sc1, sh1 = _gn_coeffs(sum1[:, 0, :], sumsq1[:, 0, :], gn1_g, gn1_b)
        X2, sum2, sumsq2 = _mlp1(X1, sc1, sh1, W2T, b2)
        sc2, sh2 = _gn_coeffs(sum2[:, 0, :], sumsq2[:, 0, :], gn2_g, gn2_b)
        outs.append(_final(X2, sc2, sh2))

    feat1_new = jnp.transpose(outs[0], (0, 2, 1))
    feat2_new = jnp.transpose(outs[1], (0, 2, 1))
    return (feat1_new, feat2_new)


# speedup vs baseline: 1.0310x; 1.0153x over previous
import jax
import jax.numpy as jnp
from jax import lax
from jax.experimental import pallas as pl
from jax.experimental.pallas import tpu as pltpu
from jax.experimental.pallas import tpu_sc as plsc

N = 2048
C = 128
CK = 64
K = 16
S = 32
NG = 8
NCOMBO = 4
RQ = 128
RSB = 2048
QB = RSB // S
GW = 128


def _prep_body(fq_ref, pq_ref, fr_ref, pr_ref, wq_ref, wr_ref, wp_ref,
               bq_ref, br_ref, q_ref, r_ref):
    fq = fq_ref[0]
    fr = fr_ref[0]
    pq = pq_ref[0]
    pr = pr_ref[0]
    posq = (pq[:, 0:1] * wp_ref[0:1, :] + pq[:, 1:2] * wp_ref[1:2, :]
            + pq[:, 2:3] * wp_ref[2:3, :])
    posr = (pr[:, 0:1] * wp_ref[0:1, :] + pr[:, 1:2] * wp_ref[1:2, :]
            + pr[:, 2:3] * wp_ref[2:3, :])
    q_ref[0] = jnp.dot(fq, wq_ref[...], preferred_element_type=jnp.float32) \
        + posq + bq_ref[...]
    r_ref[0] = jnp.dot(fr, wr_ref[...], preferred_element_type=jnp.float32) \
        - posr + br_ref[...]


def _prep(Fq, Pq, Fr, Pr, WqT, WrT, WpT, bq, br):
    RB = 512
    nc = Fq.shape[0]
    grid = (nc, N // RB)
    return pl.pallas_call(
        _prep_body,
        grid=grid,
        compiler_params=pltpu.CompilerParams(
            dimension_semantics=("parallel", "parallel")),
        in_specs=[
            pl.BlockSpec((1, RB, C), lambda c, n: (c, n, 0)),
            pl.BlockSpec((1, RB, 3), lambda c, n: (c, n, 0)),
            pl.BlockSpec((1, RB, C), lambda c, n: (c, n, 0)),
            pl.BlockSpec((1, RB, 3), lambda c, n: (c, n, 0)),
            pl.BlockSpec((C, C), lambda c, n: (0, 0)),
            pl.BlockSpec((C, C), lambda c, n: (0, 0)),
            pl.BlockSpec((3, C), lambda c, n: (0, 0)),
            pl.BlockSpec((1, C), lambda c, n: (0, 0)),
            pl.BlockSpec((1, C), lambda c, n: (0, 0)),
        ],
        out_specs=[
            pl.BlockSpec((1, RB, C), lambda c, n: (c, n, 0)),
            pl.BlockSpec((1, RB, C), lambda c, n: (c, n, 0)),
        ],
        out_shape=[
            jax.ShapeDtypeStruct((nc, N, C), jnp.float32),
            jax.ShapeDtypeStruct((nc, N, C), jnp.float32),
        ],
    )(Fq, Pq, Fr, Pr, WqT, WrT, WpT, bq, br)


NCH = 128
CW = N // NCH
RND = 5


def _extract_min16(d, acc, lane_off, lane32):
    iota_j = lax.broadcasted_iota(jnp.int32, d.shape, 1)
    for k in range(K):
        idx = jnp.argmin(d, axis=1).astype(jnp.int32)[:, None]
        acc = jnp.where(lane32 == (k + lane_off), idx, acc)
        d = jnp.where(iota_j == idx, jnp.float32(jnp.inf), d)
    return acc


def _fold_min(x):
    w = x.shape[1]
    while w > NCH:
        w //= 2
        x = jnp.minimum(x[:, :w], x[:, w:])
    return x


def _extract16_fast(d, acc, lane_off, lane32):
    iota_j = lax.broadcasted_iota(jnp.int32, (RQ, N), 1)
    vals_l, idxs_l = [], []
    for _ in range(RND):
        cm = _fold_min(d)
        cmt = jnp.tile(cm, (1, CW))
        cand = jnp.where(d == cmt, iota_j, jnp.int32(N))
        am = _fold_min(cand)
        amt = jnp.tile(am, (1, CW))
        d = jnp.where(cand == amt, jnp.float32(jnp.inf), d)
        vals_l.append(cm)
        idxs_l.append(am)
    vals = jnp.concatenate(vals_l, axis=1)
    idxs = jnp.concatenate(idxs_l, axis=1)
    for k in range(K):
        m = jnp.min(vals, axis=1, keepdims=True)
        c2 = jnp.where(vals == m, idxs, jnp.int32(N))
        jmin = jnp.min(c2, axis=1, keepdims=True)
        acc = jnp.where(lane32 == (k + lane_off), jmin, acc)
        vals = jnp.where((vals == m) & (idxs == jmin),
                         jnp.float32(jnp.inf), vals)
    exhausted = jnp.any(jnp.isinf(vals[:, (RND - 1) * NCH:]))
    return acc, exhausted


def _topk(Aq, Bc, Xq, Xc, base):
    nc = Aq.shape[0]

    def body(aq_ref, bc_ref, xq_ref, xc_ref, out_ref):
        combo = pl.program_id(0)
        xq = xq_ref[0]
        dsq = jnp.zeros((RQ, N), jnp.float32)
        for cdim in range(3):
            diff = xq[:, cdim:cdim + 1] - xc_ref[0, cdim:cdim + 1, :]
            dsq = dsq + diff * diff
        dpoint = jnp.sqrt(dsq)
        a = aq_ref[0]
        an = a / (jnp.sqrt(jnp.sum(a * a, axis=1, keepdims=True)) + 1e-8)
        b = bc_ref[0]
        bn = b / (jnp.sqrt(jnp.sum(b * b, axis=1, keepdims=True)) + 1e-8)
        sim = lax.dot_general(an, bn, (((1,), (1,)), ((), ())),
                              preferred_element_type=jnp.float32)
        dfeat = 1.0 - sim
        lane32 = lax.broadcasted_iota(jnp.int32, (RQ, S), 1)
        acc = jnp.zeros((RQ, S), jnp.int32)
        acc, bad_p = _extract16_fast(dpoint, acc, 0, lane32)
        acc, bad_f = _extract16_fast(dfeat, acc, K, lane32)
        out_ref[0] = acc + (combo + base) * N

        @pl.when(bad_p | bad_f)
        def _():
            acc2 = jnp.zeros((RQ, S), jnp.int32)
            acc2 = _extract_min16(dpoint, acc2, 0, lane32)
            acc2 = _extract_min16(dfeat, acc2, K, lane32)
            out_ref[0] = acc2 + (combo + base) * N

    grid = (nc, N // RQ)
    return pl.pallas_call(
        body,
        grid=grid,
        compiler_params=pltpu.CompilerParams(
            dimension_semantics=("parallel", "parallel")),
        in_specs=[
            pl.BlockSpec((1, RQ, CK), lambda c, n: (c, n, 0)),
            pl.BlockSpec((1, N, CK), lambda c, n: (c, 0, 0)),
            pl.BlockSpec((1, RQ, 3), lambda c, n: (c, n, 0)),
            pl.BlockSpec((1, 3, N), lambda c, n: (c, 0, 0)),
        ],
        out_specs=pl.BlockSpec((1, RQ, S), lambda c, n: (c, n, 0)),
        out_shape=jax.ShapeDtypeStruct((nc, N, S), jnp.int32),
    )(Aq, Bc, Xq, Xc)


def _sc_gather(qflat, idxflat):
    num_idx = idxflat.shape[0]
    idx2 = idxflat.reshape(1, num_idx)
    mesh = plsc.VectorSubcoreMesh(core_axis_name="c", subcore_axis_name="s")

    @pl.kernel(out_type=jax.ShapeDtypeStruct((num_idx, C), jnp.float32),
               mesh=mesh)
    def k(x_hbm, i_hbm, o_hbm):
        def body(i_vmem, o_vmem):
            pltpu.sync_copy(x_hbm.at[i_vmem.at[0]], o_vmem)

        pltpu.emit_pipeline(
            body,
            grid=(num_idx // GW,),
            in_specs=[pl.BlockSpec((1, GW), index_map=lambda i: (0, i))],
            out_specs=[pl.BlockSpec((GW, C), index_map=lambda i: (i, 0))],
            core_axis_name=("c", "s"),
            dimension_semantics=(pltpu.PARALLEL,),
        )(i_hbm, o_hbm)

    return k(qflat, idx2)


def _x0_block(g_ref, r_ref):
    g = g_ref[0]
    r = r_ref[0]
    rrep = jnp.broadcast_to(r[:, None, :], (QB, S, C)).reshape(RSB, C)
    return g + rrep


def _accum_stats(x, sum_ref, sumsq_ref, nblk_id):
    ps = jnp.sum(x, axis=0, keepdims=True)
    psq = jnp.sum(x * x, axis=0, keepdims=True)

    @pl.when(nblk_id == 0)
    def _():
        sum_ref[0] = jnp.zeros((1, C), jnp.float32)
        sumsq_ref[0] = jnp.zeros((1, C), jnp.float32)

    sum_ref[0] += ps
    sumsq_ref[0] += psq


def _stats0_body(g_ref, r_ref, sum_ref, sumsq_ref):
    x0 = _x0_block(g_ref, r_ref)
    _accum_stats(x0, sum_ref, sumsq_ref, pl.program_id(1))


def _stats0(Gact, Rt):
    nc = Gact.shape[0]
    grid = (nc, (N * S) // RSB)
    return pl.pallas_call(
        _stats0_body,
        grid=grid,
        compiler_params=pltpu.CompilerParams(
            dimension_semantics=("parallel", "arbitrary")),
        in_specs=[
            pl.BlockSpec((1, RSB, C), lambda c, n: (c, n, 0)),
            pl.BlockSpec((1, QB, C), lambda c, n: (c, n, 0)),
        ],
        out_specs=[
            pl.BlockSpec((1, 1, C), lambda c, n: (c, 0, 0)),
            pl.BlockSpec((1, 1, C), lambda c, n: (c, 0, 0)),
        ],
        out_shape=[
            jax.ShapeDtypeStruct((nc, 1, C), jnp.float32),
            jax.ShapeDtypeStruct((nc, 1, C), jnp.float32),
        ],
    )(Gact, Rt)


def _lrelu(x):
    return jnp.where(x >= 0, x, 0.1 * x)


def _mlp0_body(g_ref, r_ref, sc_ref, sh_ref, w_ref, b_ref,
               x1_ref, sum_ref, sumsq_ref):
    x0 = _x0_block(g_ref, r_ref)
    y0 = _lrelu(x0 * sc_ref[0] + sh_ref[0])
    x1 = jnp.dot(y0, w_ref[...], preferred_element_type=jnp.float32) \
        + b_ref[...]
    x1_ref[0] = x1
    _accum_stats(x1, sum_ref, sumsq_ref, pl.program_id(1))


def _mlp0(Gact, Rt, scale0, shift0, W1T, b1):
    nc = Gact.shape[0]
    grid = (nc, (N * S) // RSB)
    return pl.pallas_call(
        _mlp0_body,
        grid=grid,
        compiler_params=pltpu.CompilerParams(
            dimension_semantics=("parallel", "arbitrary")),
        in_specs=[
            pl.BlockSpec((1, RSB, C), lambda c, n: (c, n, 0)),
            pl.BlockSpec((1, QB, C), lambda c, n: (c, n, 0)),
            pl.BlockSpec((1, 1, C), lambda c, n: (c, 0, 0)),
            pl.BlockSpec((1, 1, C), lambda c, n: (c, 0, 0)),
            pl.BlockSpec((C, C), lambda c, n: (0, 0)),
            pl.BlockSpec((1, C), lambda c, n: (0, 0)),
        ],
        out_specs=[
            pl.BlockSpec((1, RSB, C), lambda c, n: (c, n, 0)),
            pl.BlockSpec((1, 1, C), lambda c, n: (c, 0, 0)),
            pl.BlockSpec((1, 1, C), lambda c, n: (c, 0, 0)),
        ],
        out_shape=[
            jax.ShapeDtypeStruct((nc, N * S, C), jnp.float32),
            jax.ShapeDtypeStruct((nc, 1, C), jnp.float32),
            jax.ShapeDtypeStruct((nc, 1, C), jnp.float32),
        ],
    )(Gact, Rt, scale0, shift0, W1T, b1)


def _mlp1_body(x_ref, sc_ref, sh_ref, w_ref, b_ref,
               x2_ref, sum_ref, sumsq_ref):
    y = _lrelu(x_ref[0] * sc_ref[0] + sh_ref[0])
    x2 = jnp.dot(y, w_ref[...], preferred_element_type=jnp.float32) \
        + b_ref[...]
    x2_ref[0] = x2
    _accum_stats(x2, sum_ref, sumsq_ref, pl.program_id(1))


def _mlp1(X1, scale1, shift1, W2T, b2):
    nc = X1.shape[0]
    grid = (nc, (N * S) // RSB)
    return pl.pallas_call(
        _mlp1_body,
        grid=grid,
        compiler_params=pltpu.CompilerParams(
            dimension_semantics=("parallel", "arbitrary")),
        in_specs=[
            pl.BlockSpec((1, RSB, C), lambda c, n: (c, n, 0)),
            pl.BlockSpec((1, 1, C), lambda c, n: (c, 0, 0)),
            pl.BlockSpec((1, 1, C), lambda c, n: (c, 0, 0)),
            pl.BlockSpec((C, C), lambda c, n: (0, 0)),
            pl.BlockSpec((1, C), lambda c, n: (0, 0)),
        ],
        out_specs=[
            pl.BlockSpec((1, RSB, C), lambda c, n: (c, n, 0)),
            pl.BlockSpec((1, 1, C), lambda c, n: (c, 0, 0)),
            pl.BlockSpec((1, 1, C), lambda c, n: (c, 0, 0)),
        ],
        out_shape=[
            jax.ShapeDtypeStruct((nc, N * S, C), jnp.float32),
            jax.ShapeDtypeStruct((nc, 1, C), jnp.float32),
            jax.ShapeDtypeStruct((nc, 1, C), jnp.float32),
        ],
    )(X1, scale1, shift1, W2T, b2)


def _final_body(x_ref, sc_ref, sh_ref, out_ref):
    y = _lrelu(x_ref[0] * sc_ref[0] + sh_ref[0])
    out_ref[0] = jnp.max(y.reshape(QB, S, C), axis=1)


def _final(X2, scale2, shift2):
    nc = X2.shape[0]
    grid = (nc, (N * S) // RSB)
    return pl.pallas_call(
        _final_body,
        grid=grid,
        compiler_params=pltpu.CompilerParams(
            dimension_semantics=("parallel", "parallel")),
        in_specs=[
            pl.BlockSpec((1, RSB, C), lambda c, n: (c, n, 0)),
            pl.BlockSpec((1, 1, C), lambda c, n: (c, 0, 0)),
            pl.BlockSpec((1, 1, C), lambda c, n: (c, 0, 0)),
        ],
        out_specs=pl.BlockSpec((1, QB, C), lambda c, n: (c, n, 0)),
        out_shape=jax.ShapeDtypeStruct((nc, N, C), jnp.float32),
    )(X2, scale2, shift2)


def _gn_coeffs(sums, sumsqs, gamma, beta):
    nc = sums.shape[0]
    M = float(N * S * (C // NG))
    gsum = sums.reshape(nc, NG, C // NG).sum(axis=2)
    gsumsq = sumsqs.reshape(nc, NG, C // NG).sum(axis=2)
    mean = gsum / M
    var = gsumsq / M - mean * mean
    inv = 1.0 / jnp.sqrt(var + 1e-5)
    mean_c = jnp.repeat(mean, C // NG, axis=1)
    inv_c = jnp.repeat(inv, C // NG, axis=1)
    scale = inv_c * gamma[None, :]
    shift = beta[None, :] - mean_c * scale
    return scale[:, None, :], shift[:, None, :]


def kernel(pc1, pc2, feat1, feat2, knn1, knn2, W_t11, b_t11, W_t22, b_t22,
           W_pos, b_pos, gn0_g, gn0_b, W_m1, b_m1, gn1_g, gn1_b,
           W_m2, b_m2, gn2_g, gn2_b):
    f32 = jnp.float32
    f1T = jnp.transpose(feat1, (0, 2, 1))
    f2T = jnp.transpose(feat2, (0, 2, 1))
    p1T = jnp.transpose(pc1, (0, 2, 1))
    p2T = jnp.transpose(pc2, (0, 2, 1))
    k1T = jnp.transpose(knn1, (0, 2, 1))
    k2T = jnp.transpose(knn2, (0, 2, 1))

    Fq = jnp.concatenate([f2T, f1T], axis=0)
    Pq = jnp.concatenate([p2T, p1T], axis=0)
    Fr = jnp.concatenate([f1T, f2T], axis=0)
    Pr = jnp.concatenate([p1T, p2T], axis=0)
    Aq = jnp.concatenate([k1T, k2T], axis=0)
    Bc = jnp.concatenate([k2T, k1T], axis=0)
    Xq = Pr
    Xc = jnp.concatenate([pc2, pc1], axis=0)

    Q, Rt = _prep(Fq, Pq, Fr, Pr,
                  W_t22.T.astype(f32), W_t11.T.astype(f32),
                  W_pos.T.astype(f32),
                  b_t22.reshape(1, C), b_t11.reshape(1, C) + b_pos.reshape(1, C))
    Qflat = Q.reshape(NCOMBO * N, C)

    W1T = W_m1.T.astype(f32)
    W2T = W_m2.T.astype(f32)
    b1 = b_m1.reshape(1, C)
    b2 = b_m2.reshape(1, C)

    outs = []
    gacts = []
    for p in (0, 1):
        sl = slice(2 * p, 2 * p + 2)
        knn_idx = _topk(Aq[sl], Bc[sl], Xq[sl], Xc[sl], base=2 * p)
        gacts.append(_sc_gather(Qflat, knn_idx.reshape(2 * N * S))
                     .reshape(2, N * S, C))
    for p in (0, 1):
        sl = slice(2 * p, 2 * p + 2)
        Gact = gacts[p]
        Rp = Rt[sl]
        sum0, sumsq0 = _stats0(Gact, Rp)
        sc0, sh0 = _gn_coeffs(sum0[:, 0, :], sumsq0[:, 0, :], gn0_g, gn0_b)
        X1, sum1, sumsq1 = _mlp0(Gact, Rp, sc0, sh0, W1T, b1)
        sc1, sh1 = _gn_coeffs(sum1[:, 0, :], sumsq1[:, 0, :], gn1_g, gn1_b)
        X2, sum2, sumsq2 = _mlp1(X1, sc1, sh1, W2T, b2)
        sc2, sh2 = _gn_coeffs(sum2[:, 0, :], sumsq2[:, 0, :], gn2_g, gn2_b)
        outs.append(_final(X2, sc2, sh2))

    feat1_new = jnp.transpose(outs[0], (0, 2, 1))
    feat2_new = jnp.transpose(outs[1], (0, 2, 1))
    return (feat1_new, feat2_new)
